# Initial kernel scaffold; baseline (speedup 1.0000x reference)
import jax
import jax.numpy as jnp
from jax.experimental import pallas as pl


def kernel(x_in, codebook):
    raise NotImplementedError("write your pallas kernel here")


# speedup vs baseline: 1.2333x; 1.2333x over previous
import functools

import jax
import jax.numpy as jnp
from jax import lax
from jax.experimental import pallas as pl
from jax.experimental.pallas import tpu as pltpu
from jax.experimental.pallas import tpu_sc as plsc

_B, _D, _N = 8, 256, 1024
_K = 8192
_M = _B * _N
_TM = 256
_NT = _M // _TM

_NC, _NS = 2, 16
_NW = _NC * _NS
_ROWS_PER_W = _M // _NW
_HALF = _ROWS_PER_W // 2


def _nearest_code_body(xt_ref, x2_ref, e2_ref, cb_ref, idx_ref):
    s = lax.dot_general(
        xt_ref[...], cb_ref[...],
        (((1,), (1,)), ((), ())),
        preferred_element_type=jnp.float32,
    )
    d = (x2_ref[0, 0, :][:, None] + e2_ref[0, :][None, :]) - 2.0 * s
    d = jnp.maximum(d, 0.0)
    idx_ref[0, 0, :] = jnp.argmin(d, axis=1).astype(jnp.int32)


def _nearest_codes(xt, x2, e2, codebook):
    return pl.pallas_call(
        _nearest_code_body,
        grid=(_NT,),
        in_specs=[
            pl.BlockSpec((_TM, _D), lambda i: (i, 0)),
            pl.BlockSpec((1, 1, _TM), lambda i: (i, 0, 0)),
            pl.BlockSpec((1, _K), lambda i: (0, 0)),
            pl.BlockSpec((_K, _D), lambda i: (0, 0)),
        ],
        out_specs=pl.BlockSpec((1, 1, _TM), lambda i: (i, 0, 0)),
        out_shape=jax.ShapeDtypeStruct((_NT, 1, _TM), jnp.int32),
    )(xt, x2.reshape(_NT, 1, _TM), e2.reshape(1, _K), codebook)


def _sc_gather_body(idx_hbm, table_hbm, out_hbm, idx_v, rows_v, sem0, sem1):
    wid = lax.axis_index("s") * _NC + lax.axis_index("c")
    base = wid * _ROWS_PER_W
    pltpu.sync_copy(idx_hbm.at[pl.ds(base, _ROWS_PER_W)], idx_v)
    c0 = pltpu.async_copy(
        table_hbm.at[idx_v.at[pl.ds(0, _HALF)]], rows_v.at[pl.ds(0, _HALF)], sem0)
    c1 = pltpu.async_copy(
        table_hbm.at[idx_v.at[pl.ds(_HALF, _HALF)]], rows_v.at[pl.ds(_HALF, _HALF)], sem1)
    c0.wait()
    c1.wait()
    pltpu.sync_copy(rows_v, out_hbm.at[pl.ds(base, _ROWS_PER_W)])


@functools.cache
def _sc_gather():
    return pl.kernel(
        _sc_gather_body,
        out_type=jax.ShapeDtypeStruct((_M, _D), jnp.float32),
        mesh=plsc.VectorSubcoreMesh(core_axis_name="c", subcore_axis_name="s",
                                    num_cores=_NC, num_subcores=_NS),
        scratch_types=[
            pltpu.VMEM((_ROWS_PER_W,), jnp.int32),
            pltpu.VMEM((_ROWS_PER_W, _D), jnp.float32),
            pltpu.SemaphoreType.DMA,
            pltpu.SemaphoreType.DMA,
        ],
    )


def kernel(x_in, codebook):
    xt3 = jnp.transpose(x_in, (0, 2, 1))
    x2 = jnp.sum(xt3 * xt3, axis=-1)
    e2 = jnp.sum(codebook * codebook, axis=-1)
    idx = _nearest_codes(xt3.reshape(_M, _D), x2, e2, codebook)
    q = _sc_gather()(idx.reshape(_M), codebook)
    return q.reshape(_B, _N, _D)


# speedup vs baseline: 1.4518x; 1.1771x over previous
import functools

import jax
import jax.numpy as jnp
from jax import lax
from jax.experimental import pallas as pl
from jax.experimental.pallas import tpu as pltpu
from jax.experimental.pallas import tpu_sc as plsc

_B, _D, _N = 8, 256, 1024
_K = 8192
_M = _B * _N
_TM = 256
_NT = _M // _TM

_NC, _NS = 2, 16
_NW = _NC * _NS
_ROWS_PER_W = _M // _NW
_HALF = _ROWS_PER_W // 2


def _nearest_code_body(x_ref, x2_ref, e2_ref, cb_ref, idx_ref):
    xd = x_ref[0] + x_ref[0]
    s2 = lax.dot_general(
        xd, cb_ref[...],
        (((0,), (1,)), ((), ())),
        preferred_element_type=jnp.float32,
    )
    d = (x2_ref[0, 0, :][:, None] + e2_ref[0, :][None, :]) - s2
    d = jnp.maximum(d, 0.0)
    idx_ref[0, 0, :] = jnp.argmin(d, axis=1).astype(jnp.int32)


_NTB = _N // _TM


def _nearest_codes(x_in, x2, e2, codebook):
    return pl.pallas_call(
        _nearest_code_body,
        grid=(_NT,),
        in_specs=[
            pl.BlockSpec((1, _D, _TM), lambda i: (i // _NTB, 0, i % _NTB)),
            pl.BlockSpec((1, 1, _TM), lambda i: (i, 0, 0)),
            pl.BlockSpec((1, _K), lambda i: (0, 0)),
            pl.BlockSpec((_K, _D), lambda i: (0, 0)),
        ],
        out_specs=pl.BlockSpec((1, 1, _TM), lambda i: (i, 0, 0)),
        out_shape=jax.ShapeDtypeStruct((_NT, 1, _TM), jnp.int32),
    )(x_in, x2.reshape(_NT, 1, _TM), e2.reshape(1, _K), codebook)


def _sc_gather_body(idx_hbm, table_hbm, out_hbm, idx_v, rows_v, sem0, sem1):
    wid = lax.axis_index("s") * _NC + lax.axis_index("c")
    base = wid * _ROWS_PER_W
    pltpu.sync_copy(idx_hbm.at[pl.ds(base, _ROWS_PER_W)], idx_v)
    c0 = pltpu.async_copy(
        table_hbm.at[idx_v.at[pl.ds(0, _HALF)]], rows_v.at[pl.ds(0, _HALF)], sem0)
    c1 = pltpu.async_copy(
        table_hbm.at[idx_v.at[pl.ds(_HALF, _HALF)]], rows_v.at[pl.ds(_HALF, _HALF)], sem1)
    c0.wait()
    c1.wait()
    pltpu.sync_copy(rows_v, out_hbm.at[pl.ds(base, _ROWS_PER_W)])


@functools.cache
def _sc_gather():
    return pl.kernel(
        _sc_gather_body,
        out_type=jax.ShapeDtypeStruct((_M, _D), jnp.float32),
        mesh=plsc.VectorSubcoreMesh(core_axis_name="c", subcore_axis_name="s",
                                    num_cores=_NC, num_subcores=_NS),
        scratch_types=[
            pltpu.VMEM((_ROWS_PER_W,), jnp.int32),
            pltpu.VMEM((_ROWS_PER_W, _D), jnp.float32),
            pltpu.SemaphoreType.DMA,
            pltpu.SemaphoreType.DMA,
        ],
    )


def kernel(x_in, codebook):
    xt3 = jnp.transpose(x_in, (0, 2, 1))
    x2 = jnp.sum(xt3 * xt3, axis=-1)
    e2 = jnp.sum(codebook * codebook, axis=-1)
    idx = _nearest_codes(x_in, x2, e2, codebook)
    q = _sc_gather()(idx.reshape(_M), codebook)
    return q.reshape(_B, _N, _D)
